# Initial kernel scaffold; baseline (speedup 1.0000x reference)
import jax
import jax.numpy as jnp
from jax.experimental import pallas as pl


def kernel(s, z, edge_index, r_rots, r_trans, mask, W_q, b_q, W_kv, b_kv, W_qp, b_qp, W_kvp, b_kvp, W_b, b_b, W_dz, b_dz, head_weights, W_out, b_out, W_rbf, b_rbf):
    raise NotImplementedError("write your pallas kernel here")


# speedup vs baseline: 14.0542x; 14.0542x over previous
import functools

import jax
import jax.numpy as jnp
import numpy as np
from jax import lax
from jax.experimental import pallas as pl
from jax.experimental.pallas import tpu as pltpu
from jax.experimental.pallas import tpu_sc as plsc

B, N, K = 1, 4096, 32
Cs, Cz, Ch, H, Pq, Pv = 384, 128, 16, 12, 4, 8
E = N * K
INF, EPS = 1e5, 1e-8

NB_PREP = 512
BN = 64
EB = BN * K

F32 = np.float32


def _sel(n_in, group):
    m = np.zeros((n_in, n_in // group), F32)
    m[np.arange(n_in), np.arange(n_in) // group] = 1.0
    return m


_S1 = _sel(192, 16)
_S2 = _sel(48, 4)
_E192 = _sel(192, 16).T
_E96 = _sel(96, 8).T
_E384 = _sel(384, 32).T
_EZ = np.zeros((32, 384), F32)
_EZ[np.arange(384) % 32, np.arange(384)] = 1.0

_KV_PERM = np.empty(384, np.int64)
for _n in range(384):
    if _n < 192:
        _h, _c = _n // 16, _n % 16
        _KV_PERM[_n] = _h * 32 + _c
    else:
        _h, _c = (_n - 192) // 16, (_n - 192) % 16
        _KV_PERM[_n] = _h * 32 + 16 + _c

_KVP_PERM = np.empty(432, np.int64)
for _n in range(432):
    _j, _r = _n // 144, _n % 144
    if _r < 48:
        _h, _p = _r // 4, _r % 4
    else:
        _h, _p = (_r - 48) // 8, 4 + (_r - 48) % 8
    _KVP_PERM[_n] = _j * 144 + _h * 12 + _p

_WOUT_PERM = np.empty(960, np.int64)
for _n in range(960):
    if _n < 192:
        _WOUT_PERM[_n] = _n
    elif _n < 480:
        _d, _hp = (_n - 192) // 96, (_n - 192) % 96
        _WOUT_PERM[_n] = 192 + _hp * 3 + _d
    else:
        _WOUT_PERM[_n] = _n


def _prep_body(s_ref, z_ref, r9_ref, t_ref, m_ref,
               wq_ref, bq_ref, wkv_ref, bkv_ref, wqp_ref, bqp_ref,
               wkvp_ref, bkvp_ref, wz_ref, bz_ref,
               q_out, qpts_out, taba_out, tabb_out, zb_out):
    s = s_ref[...]
    q_out[...] = s @ wq_ref[...] + bq_ref[...]
    taba_out[...] = s @ wkv_ref[...] + bkv_ref[...]
    r9 = r9_ref[...]
    t = t_ref[...]

    qp = s @ wqp_ref[...] + bqp_ref[...]
    qcomps = []
    for i in range(3):
        acc = t[:, i:i + 1]
        for j in range(3):
            acc = acc + r9[:, 3 * i + j:3 * i + j + 1] * qp[:, j * 48:(j + 1) * 48]
        qcomps.append(acc)
    qpts_out[...] = jnp.concatenate(qcomps, axis=1)

    kvp = s @ wkvp_ref[...] + bkvp_ref[...]
    pieces = []
    for i in range(3):
        acck = t[:, i:i + 1]
        accv = t[:, i:i + 1]
        for j in range(3):
            r = r9[:, 3 * i + j:3 * i + j + 1]
            acck = acck + r * kvp[:, j * 144:j * 144 + 48]
            accv = accv + r * kvp[:, j * 144 + 48:(j + 1) * 144]
        pieces += [acck, accv]
    pieces += [t, m_ref[...], jnp.zeros((s.shape[0], 76), jnp.float32)]
    tabb_out[...] = jnp.concatenate(pieces, axis=1)

    zb_out[...] = z_ref[...] @ wz_ref[...] + bz_ref[...]


def _prep(s2, z2, r9, t2, m2, wqt, bq2, wkvt, bkv2, wqpt, bqp2, wkvpt, bkvp2,
          wzt, bz2):
    nsteps = N // NB_PREP
    ezb = NB_PREP * K
    full = lambda i: (0, 0)
    row = lambda i: (i, 0)
    return pl.pallas_call(
        _prep_body,
        grid=(nsteps,),
        in_specs=[
            pl.BlockSpec((NB_PREP, Cs), row),
            pl.BlockSpec((ezb, Cz), row),
            pl.BlockSpec((NB_PREP, 9), row),
            pl.BlockSpec((NB_PREP, 3), row),
            pl.BlockSpec((NB_PREP, 1), row),
            pl.BlockSpec((Cs, 192), full), pl.BlockSpec((1, 192), full),
            pl.BlockSpec((Cs, 384), full), pl.BlockSpec((1, 384), full),
            pl.BlockSpec((Cs, 144), full), pl.BlockSpec((1, 144), full),
            pl.BlockSpec((Cs, 432), full), pl.BlockSpec((1, 432), full),
            pl.BlockSpec((Cz, 44), full), pl.BlockSpec((1, 44), full),
        ],
        out_specs=[
            pl.BlockSpec((NB_PREP, 192), row),
            pl.BlockSpec((NB_PREP, 144), row),
            pl.BlockSpec((NB_PREP, 384), row),
            pl.BlockSpec((NB_PREP, 512), row),
            pl.BlockSpec((ezb, 44), row),
        ],
        out_shape=[
            jax.ShapeDtypeStruct((N, 192), jnp.float32),
            jax.ShapeDtypeStruct((N, 144), jnp.float32),
            jax.ShapeDtypeStruct((N, 384), jnp.float32),
            jax.ShapeDtypeStruct((N, 512), jnp.float32),
            jax.ShapeDtypeStruct((E, 44), jnp.float32),
        ],
    )(s2, z2, r9, t2, m2, wqt, bq2, wkvt, bkv2, wqpt, bqp2, wkvpt, bkvp2,
      wzt, bz2)


_SC_CHUNK = 128
_NW = 32


def _sc_gather_body(taba, tabb, ei, ag, bg, idx_c, bufa, bufb, sem):
    wid = lax.axis_index("s") * 2 + lax.axis_index("c")
    per_w = E // _NW
    base = wid * per_w

    def chunk(c, carry):
        off = base + c * _SC_CHUNK
        pltpu.sync_copy(ei.at[pl.ds(off, _SC_CHUNK)], idx_c)
        cpa = pltpu.async_copy(taba.at[idx_c], bufa, sem)
        cpb = pltpu.async_copy(tabb.at[idx_c], bufb, sem)
        cpa.wait()
        cpb.wait()
        pltpu.sync_copy(bufa, ag.at[pl.ds(off, _SC_CHUNK)])
        pltpu.sync_copy(bufb, bg.at[pl.ds(off, _SC_CHUNK)])
        return carry

    lax.fori_loop(0, per_w // _SC_CHUNK, chunk, 0)


def _sc_gather(taba, tabb, ei):
    mesh = plsc.VectorSubcoreMesh(core_axis_name="c", subcore_axis_name="s")
    fn = functools.partial(
        pl.kernel,
        mesh=mesh,
        out_type=[
            jax.ShapeDtypeStruct((E, 384), jnp.float32),
            jax.ShapeDtypeStruct((E, 512), jnp.float32),
        ],
        scratch_types=[
            pltpu.VMEM((_SC_CHUNK,), jnp.int32),
            pltpu.VMEM((_SC_CHUNK, 384), jnp.float32),
            pltpu.VMEM((_SC_CHUNK, 512), jnp.float32),
            pltpu.SemaphoreType.DMA,
        ],
    )(_sc_gather_body)
    return fn(taba, tabb, ei)


def _consume_body(q_ref, qpts_ref, r9_ref, t_ref, m_ref, zb_ref, ag_ref,
                  bg_ref, hw_ref, wrbf_ref, brbf_ref, s1_ref, s2_ref,
                  e192_ref, e96_ref, e384_ref, ez_ref,
                  wo_o_ref, wo_l0_ref, wo_l1_ref, wo_l2_ref, wo_n_ref,
                  wo_p_ref, bo_ref, out_ref):
    def bcast(x):
        c = x.shape[1]
        return jnp.broadcast_to(x[:, None, :], (BN, K, c)).reshape(EB, c)

    def ksum(x):
        c = x.shape[1]
        return x.reshape(BN, K, c).sum(axis=1)

    ag = ag_ref[...]
    kg, vg = ag[:, :192], ag[:, 192:]
    qe = bcast(q_ref[...])
    a = ((qe * kg) @ s1_ref[...]) * np.sqrt(1.0 / (3 * Ch))

    zb = zb_ref[...]
    a = a + np.sqrt(1.0 / 3) * zb[:, :12]

    bg = bg_ref[...]
    qpe = bcast(qpts_ref[...])
    d2 = None
    for i in range(3):
        diff = qpe[:, i * 48:(i + 1) * 48] - bg[:, i * 144:i * 144 + 48]
        sq = diff * diff
        d2 = sq if d2 is None else d2 + sq
    hw = jnp.log1p(jnp.exp(hw_ref[...])) * np.sqrt(1.0 / (3 * (Pq * 9.0 / 2)))
    a = a - 0.5 * ((d2 @ s2_ref[...]) * hw)

    t = t_ref[...]
    te = bcast(t)
    tg = bg[:, 432:435]
    dd = te - tg
    dist = jnp.sqrt(jnp.sum(dd * dd, axis=1, keepdims=True) + EPS)
    cent = lax.broadcasted_iota(jnp.int32, (1, 20), 1).astype(jnp.float32) * (
        20.0 / 19.0)
    rbf = jnp.exp(-((dist - cent) ** 2))
    a = a + (rbf @ wrbf_ref[...] + brbf_ref[...])

    me = bcast(m_ref[...])
    a = a + INF * (me * bg[:, 435:436] - 1.0)

    a3 = a.reshape(BN, K, 12)
    amax = jnp.max(a3, axis=1, keepdims=True)
    p = jnp.exp(a3 - amax)
    w = (p / jnp.sum(p, axis=1, keepdims=True)).reshape(EB, 12)

    o = ksum((w @ e192_ref[...]) * vg)

    w96 = w @ e96_ref[...]
    opts = []
    for i in range(3):
        vi = bg[:, i * 144 + 48:(i + 1) * 144]
        opts.append(ksum(w96 * vi) - t[:, i:i + 1])
    r9 = r9_ref[...]
    locs = []
    for i in range(3):
        acc = None
        for j in range(3):
            term = r9[:, 3 * j + i:3 * j + i + 1] * opts[j]
            acc = term if acc is None else acc + term
        locs.append(acc)
    nrm = jnp.sqrt(locs[0] ** 2 + locs[1] ** 2 + locs[2] ** 2 + EPS)

    opair = ksum((w @ e384_ref[...]) * (zb[:, 12:44] @ ez_ref[...]))

    out = (o @ wo_o_ref[...] + locs[0] @ wo_l0_ref[...]
           + locs[1] @ wo_l1_ref[...] + locs[2] @ wo_l2_ref[...]
           + nrm @ wo_n_ref[...] + opair @ wo_p_ref[...] + bo_ref[...])
    out_ref[...] = out


def _consume(q, qpts, r9, t2, m2, zb, ag, bg, hw2, wrbft, brbf2,
             s1, s2c, e192, e96, e384, ez, wo_o, wo_l0, wo_l1, wo_l2,
             wo_n, wo_p, bo2):
    nsteps = N // BN
    full = lambda i: (0, 0)
    row = lambda i: (i, 0)
    return pl.pallas_call(
        _consume_body,
        grid=(nsteps,),
        in_specs=[
            pl.BlockSpec((BN, 192), row),
            pl.BlockSpec((BN, 144), row),
            pl.BlockSpec((BN, 9), row),
            pl.BlockSpec((BN, 3), row),
            pl.BlockSpec((BN, 1), row),
            pl.BlockSpec((EB, 44), row),
            pl.BlockSpec((EB, 384), row),
            pl.BlockSpec((EB, 512), row),
            pl.BlockSpec((1, 12), full),
            pl.BlockSpec((20, 1), full),
            pl.BlockSpec((1, 1), full),
            pl.BlockSpec((192, 12), full),
            pl.BlockSpec((48, 12), full),
            pl.BlockSpec((12, 192), full),
            pl.BlockSpec((12, 96), full),
            pl.BlockSpec((12, 384), full),
            pl.BlockSpec((32, 384), full),
            pl.BlockSpec((192, 384), full),
            pl.BlockSpec((96, 384), full),
            pl.BlockSpec((96, 384), full),
            pl.BlockSpec((96, 384), full),
            pl.BlockSpec((96, 384), full),
            pl.BlockSpec((384, 384), full),
            pl.BlockSpec((1, 384), full),
        ],
        out_specs=pl.BlockSpec((BN, 384), row),
        out_shape=jax.ShapeDtypeStruct((N, 384), jnp.float32),
    )(q, qpts, r9, t2, m2, zb, ag, bg, hw2, wrbft, brbf2, s1, s2c, e192,
      e96, e384, ez, wo_o, wo_l0, wo_l1, wo_l2, wo_n, wo_p, bo2)


def kernel(s, z, edge_index, r_rots, r_trans, mask,
           W_q, b_q, W_kv, b_kv, W_qp, b_qp, W_kvp, b_kvp,
           W_b, b_b, W_dz, b_dz, head_weights, W_out, b_out, W_rbf, b_rbf):
    f32 = jnp.float32
    s2 = s.reshape(N, Cs).astype(f32)
    z2 = z.reshape(E, Cz).astype(f32)
    ei = edge_index.reshape(E).astype(jnp.int32)
    r9 = r_rots.reshape(N, 9).astype(f32)
    t2 = r_trans.reshape(N, 3).astype(f32)
    m2 = mask.reshape(N, 1).astype(f32)

    wqt = W_q.T.astype(f32)
    bq2 = b_q.reshape(1, 192).astype(f32)
    wkvt = W_kv[_KV_PERM].T.astype(f32)
    bkv2 = b_kv[_KV_PERM].reshape(1, 384).astype(f32)
    wqpt = W_qp.T.astype(f32)
    bqp2 = b_qp.reshape(1, 144).astype(f32)
    wkvpt = W_kvp[_KVP_PERM].T.astype(f32)
    bkvp2 = b_kvp[_KVP_PERM].reshape(1, 432).astype(f32)
    wzt = jnp.concatenate([W_b, W_dz], axis=0).T.astype(f32)
    bz2 = jnp.concatenate([b_b, b_dz]).reshape(1, 44).astype(f32)
    hw2 = head_weights.reshape(1, 12).astype(f32)
    wrbft = W_rbf.T.astype(f32)
    brbf2 = b_rbf.reshape(1, 1).astype(f32)
    wout_p = W_out[:, _WOUT_PERM].T.astype(f32)
    bo2 = b_out.reshape(1, 384).astype(f32)

    q, qpts, taba, tabb, zb = _prep(
        s2, z2, r9, t2, m2, wqt, bq2, wkvt, bkv2, wqpt, bqp2, wkvpt, bkvp2,
        wzt, bz2)

    ag, bg = _sc_gather(taba, tabb, ei)

    out = _consume(
        q, qpts, r9, t2, m2, zb, ag, bg, hw2, wrbft, brbf2,
        jnp.asarray(_S1), jnp.asarray(_S2), jnp.asarray(_E192),
        jnp.asarray(_E96), jnp.asarray(_E384), jnp.asarray(_EZ),
        wout_p[:192], wout_p[192:288], wout_p[288:384], wout_p[384:480],
        wout_p[480:576], wout_p[576:960], bo2)

    return out.reshape(B, N, Cs)
